# Initial kernel scaffold; baseline (speedup 1.0000x reference)
import jax
import jax.numpy as jnp
from jax import lax
from jax.experimental import pallas as pl

_F = 16
_KH = _KW = 3
_H = _W = 64
_C = 16
_K = _KH * _KW * _C
_G = 8
_TMAX = 1.0
_BIG = 1e6


def _body(p_ref, j_ref, o_ref):
    P = p_ref[:, :]
    tji = P[:, :, None]
    tjj = P[:, None, :]
    ii = lax.broadcasted_iota(jnp.int32, (_K, _K), 0)
    jj = lax.broadcasted_iota(jnp.int32, (_K, _K), 1)
    tri = (jj <= ii)[None]
    Mb = (tjj < tji) | ((tjj == tji) & tri)
    Mf = Mb.astype(jnp.float32)
    Mt = Mf * tjj

    left = jnp.concatenate(
        [Mt.reshape(_G * _K, _K), Mf.reshape(_G * _K, _K)], axis=0)
    SD = lax.dot_general(
        left, j_ref[:, :], (((1,), (0,)), ((), ())),
        precision=lax.Precision.HIGHEST,
        preferred_element_type=jnp.float32)
    S = SD[: _G * _K].reshape(_G, _K, _F)
    D = SD[_G * _K:].reshape(_G, _K, _F) + 1.0
    ti = S / D

    tjnext = jnp.min(
        jnp.where(Mb, _BIG, jnp.broadcast_to(tjj, (_G, _K, _K))), axis=2)
    rank = jnp.sum(Mf, axis=2) - 1.0

    cond = (ti < tjnext[:, :, None]) & (D > 0.0)
    selr = jnp.min(
        jnp.where(cond, jnp.broadcast_to(rank[:, :, None], (_G, _K, _F)),
                  float(_K)), axis=1)
    selr = jnp.where(selr == float(_K), 0.0, selr)
    hit = rank[:, :, None] == selr[:, None, :]
    outv = jnp.sum(jnp.where(hit, ti, 0.0), axis=1)
    o_ref[:, :] = jnp.where(outv <= _TMAX, outv, _TMAX)


def kernel(tj, kernel):
    x = tj[0].astype(jnp.float32)
    xp = jnp.pad(x, ((1, 1), (1, 1), (0, 0)))
    parts = [xp[i:i + _H, j:j + _W, :] for i in range(_KH) for j in range(_KW)]
    patches = jnp.concatenate(parts, axis=-1).reshape(_H * _W, _K)
    J = kernel.reshape(_K, _F).astype(jnp.float32)

    n = _H * _W
    out = pl.pallas_call(
        _body,
        grid=(n // _G,),
        in_specs=[
            pl.BlockSpec((_G, _K), lambda i: (i, 0)),
            pl.BlockSpec((_K, _F), lambda i: (0, 0)),
        ],
        out_specs=pl.BlockSpec((_G, _F), lambda i: (i, 0)),
        out_shape=jax.ShapeDtypeStruct((n, _F), jnp.float32),
    )(patches, J)
    return out.reshape(1, _H, _W, _F).astype(jnp.float64)


# speedup vs baseline: 24.1094x; 24.1094x over previous
import jax
import jax.numpy as jnp
from jax import lax
from jax.experimental import pallas as pl

_F = 16
_KH = _KW = 3
_H = _W = 64
_C = 16
_K = _KH * _KW * _C
_G = 8
_TMAX = 1.0
_BIG = 1e6


def _z():
    return jnp.int32(0)


def _body(p_ref, j_ref, o_ref):
    P = p_ref[:, :]
    tji = P[:, :, None]
    tjj = P[:, None, :]
    ii = lax.broadcasted_iota(jnp.int32, (_K, _K), 0)
    jj = lax.broadcasted_iota(jnp.int32, (_K, _K), 1)
    tri = (jj <= ii)[None]
    Mb = (tjj < tji) | ((tjj == tji) & tri)
    Mf = Mb.astype(jnp.float32)
    Mt = Mf * tjj

    left = jnp.concatenate(
        [Mt.reshape(_G * _K, _K), Mf.reshape(_G * _K, _K)], axis=0)
    SD = lax.dot_general(
        left, j_ref[:, :], (((1,), (0,)), ((), ())),
        precision=lax.Precision.HIGHEST,
        preferred_element_type=jnp.float32)
    S = SD[: _G * _K].reshape(_G, _K, _F)
    D = SD[_G * _K:].reshape(_G, _K, _F) + 1.0
    ti = S / D

    tjnext = jnp.min(
        jnp.where(Mb, _BIG, jnp.broadcast_to(tjj, (_G, _K, _K))), axis=2)
    rank = jnp.sum(Mf, axis=2) - 1.0

    cond = (ti < tjnext[:, :, None]) & (D > 0.0)
    selr = jnp.min(
        jnp.where(cond, jnp.broadcast_to(rank[:, :, None], (_G, _K, _F)),
                  float(_K)), axis=1)
    selr = jnp.where(selr == float(_K), 0.0, selr)
    hit = rank[:, :, None] == selr[:, None, :]
    outv = jnp.sum(jnp.where(hit, ti, 0.0), axis=1)
    o_ref[:, :] = jnp.where(outv <= _TMAX, outv, _TMAX)


def kernel(tj, kernel):
    x = tj[0].astype(jnp.float32)
    xp = jnp.pad(x, ((1, 1), (1, 1), (0, 0)))
    parts = [xp[i:i + _H, j:j + _W, :] for i in range(_KH) for j in range(_KW)]
    patches = jnp.concatenate(parts, axis=-1).reshape(_H * _W, _K)
    J = kernel.reshape(_K, _F).astype(jnp.float32)

    n = _H * _W
    out = pl.pallas_call(
        _body,
        grid=(n // _G,),
        in_specs=[
            pl.BlockSpec((_G, _K), lambda i: (i, _z())),
            pl.BlockSpec((_K, _F), lambda i: (_z(), _z())),
        ],
        out_specs=pl.BlockSpec((_G, _F), lambda i: (i, _z())),
        out_shape=jax.ShapeDtypeStruct((n, _F), jnp.float32),
    )(patches, J)
    return out.reshape(1, _H, _W, _F).astype(jnp.float64)


# speedup vs baseline: 28.3584x; 1.1762x over previous
import jax
import jax.numpy as jnp
from jax import lax
from jax.experimental import pallas as pl

_F = 16
_KH = _KW = 3
_H = _W = 64
_C = 16
_K = _KH * _KW * _C
_G = 16
_TMAX = 1.0
_BIG = 1e6


def _z():
    return jnp.int32(0)


def _body(p_ref, j_ref, o_ref):
    P = p_ref[:, :]
    tji = P[:, :, None]
    tjj = P[:, None, :]
    ii = lax.broadcasted_iota(jnp.int32, (_K, _K), 0)
    jj = lax.broadcasted_iota(jnp.int32, (_K, _K), 1)
    tri = (jj <= ii)[None]
    Mb = (tjj < tji) | ((tjj == tji) & tri)
    Mf = Mb.astype(jnp.float32)

    tj_hi = tjj.astype(jnp.bfloat16)
    tj_lo = (tjj - tj_hi.astype(jnp.float32)).astype(jnp.bfloat16)
    Mbf = Mb.astype(jnp.bfloat16)
    Mt_hi = (Mbf * tj_hi).reshape(_G * _K, _K)
    Mt_lo = (Mbf * tj_lo).reshape(_G * _K, _K)
    left = jnp.concatenate(
        [Mt_hi, Mt_lo, Mbf.reshape(_G * _K, _K)], axis=0)
    O = lax.dot_general(
        left, j_ref[:, :], (((1,), (0,)), ((), ())),
        preferred_element_type=jnp.float32)
    n = _G * _K
    S = (O[:n, :_F] + O[:n, _F:] + O[n:2 * n, :_F]).reshape(_G, _K, _F)
    D = (O[2 * n:, :_F] + O[2 * n:, _F:]).reshape(_G, _K, _F) + 1.0
    ti = S / D

    tjnext = jnp.min(
        jnp.where(Mb, _BIG, jnp.broadcast_to(tjj, (_G, _K, _K))), axis=2)
    rank = jnp.sum(Mf, axis=2) - 1.0

    cond = (ti < tjnext[:, :, None]) & (D > 0.0)
    selr = jnp.min(
        jnp.where(cond, jnp.broadcast_to(rank[:, :, None], (_G, _K, _F)),
                  float(_K)), axis=1)
    selr = jnp.where(selr == float(_K), 0.0, selr)
    hit = rank[:, :, None] == selr[:, None, :]
    outv = jnp.sum(jnp.where(hit, ti, 0.0), axis=1)
    o_ref[:, :] = jnp.where(outv <= _TMAX, outv, _TMAX)


def kernel(tj, kernel):
    x = tj[0].astype(jnp.float32)
    xp = jnp.pad(x, ((1, 1), (1, 1), (0, 0)))
    parts = [xp[i:i + _H, j:j + _W, :] for i in range(_KH) for j in range(_KW)]
    patches = jnp.concatenate(parts, axis=-1).reshape(_H * _W, _K)
    Jf = kernel.reshape(_K, _F).astype(jnp.float32)
    J_hi = Jf.astype(jnp.bfloat16)
    J_lo = (Jf - J_hi.astype(jnp.float32)).astype(jnp.bfloat16)
    J = jnp.concatenate([J_hi, J_lo], axis=1)

    n = _H * _W
    out = pl.pallas_call(
        _body,
        grid=(n // _G,),
        in_specs=[
            pl.BlockSpec((_G, _K), lambda i: (i, _z())),
            pl.BlockSpec((_K, 2 * _F), lambda i: (_z(), _z())),
        ],
        out_specs=pl.BlockSpec((_G, _F), lambda i: (i, _z())),
        out_shape=jax.ShapeDtypeStruct((n, _F), jnp.float32),
    )(patches, J)
    return out.reshape(1, _H, _W, _F).astype(jnp.float64)


# speedup vs baseline: 224.4169x; 7.9136x over previous
import functools

import jax
import jax.numpy as jnp
from jax import lax
from jax.experimental import pallas as pl
from jax.experimental.pallas import tpu as pltpu
from jax.experimental.pallas import tpu_sc as plsc

_F = 16
_KH = _KW = 3
_H = _W = 64
_C = 16
_K = _KH * _KW * _C
_N = _H * _W
_NW = 32
_RPW = _N // _NW
_CH = 32
_BIG = 1e6
_L = 16


def _cmpex(a, b):
    ka, va = a
    kb, vb = b
    c = ka <= kb
    return ((jnp.where(c, ka, kb), jnp.where(c, va, vb)),
            (jnp.where(c, kb, ka), jnp.where(c, vb, va)))


def _bitonic_merge(seq):
    n = len(seq)
    d = n // 2
    while d >= 1:
        for start in range(0, n, 2 * d):
            for i in range(start, start + d):
                lo, hi = _cmpex(seq[i], seq[i + d])
                seq[i], seq[i + d] = lo, hi
        d //= 2
    return [plsc.sort_key_val(k, v) for (k, v) in seq]


def _merge(a, b):
    brev = [(lax.rev(k, (0,)), lax.rev(v, (0,))) for (k, v) in reversed(b)]
    return _bitonic_merge(a + brev)


def _sc_body(p_hbm, j_hbm, out_hbm, jv, pv, ov, skey, sidx, sem):
    del sem
    nc = jnp.int32(2)
    wid = lax.axis_index("s") * nc + lax.axis_index("c")
    base = wid * jnp.int32(_RPW)
    iota = lax.broadcasted_iota(jnp.int32, (_L,), 0)

    pltpu.sync_copy(j_hbm, jv)

    def do_chunk(ch, carry):
        rbase = base + ch * jnp.int32(_CH)
        pltpu.sync_copy(p_hbm.at[pl.ds(rbase * jnp.int32(_K), _CH * _K)], pv)

        def do_row(rr, carry2):
            runs = []
            for i in range(9):
                kv = pv[pl.ds(rr * jnp.int32(_K) + jnp.int32(16 * i), 16)]
                ks, vs = plsc.sort_key_val(kv, iota + 16 * i)
                runs.append([(ks, vs)])
            for i in range(9, 16):
                runs.append([(jnp.full((_L,), _BIG, jnp.float32),
                              jnp.zeros((_L,), jnp.int32))])
            while len(runs) > 1:
                runs = [_merge(runs[i], runs[i + 1])
                        for i in range(0, len(runs), 2)]
            srt = runs[0]
            for i in range(10):
                skey[pl.ds(16 * i, 16)] = srt[i][0]
                sidx[pl.ds(16 * i, 16)] = srt[i][1]

            kv0 = skey[pl.ds(0, 16)]
            ov0 = sidx[pl.ds(0, 16)]
            jr0 = plsc.load_gather(jv, [ov0[0] * jnp.int32(_F) + iota])
            s0 = jr0 * kv0[0]
            dm0 = jr0
            d0 = dm0 + 1.0
            ti0 = s0 / d0
            cond0 = (ti0 < kv0[1]) & (d0 > 0.0)
            out0 = ti0
            fired0 = jnp.where(cond0, jnp.float32(1.0), jnp.float32(0.0))

            def step(k, c3):
                s, dm, out, fired = c3
                kvv = skey[pl.ds(k, 16)]
                ovv = sidx[pl.ds(k, 16)]
                jr = plsc.load_gather(jv, [ovv[0] * jnp.int32(_F) + iota])
                s = s + jr * kvv[0]
                dm = dm + jr
                d = dm + 1.0
                ti = s / d
                cond = (ti < kvv[1]) & (d > 0.0)
                take = cond & (fired == 0.0)
                out = jnp.where(take, ti, out)
                fired = jnp.where(cond, jnp.float32(1.0), fired)
                return s, dm, out, fired

            _, _, out, _ = lax.fori_loop(
                jnp.int32(1), jnp.int32(_K), step, (s0, dm0, out0, fired0))
            out = jnp.where(out <= 1.0, out, 1.0)
            ov[pl.ds(rr * jnp.int32(_F), _F)] = out
            return carry2

        lax.fori_loop(jnp.int32(0), jnp.int32(_CH), do_row, jnp.int32(0))
        pltpu.sync_copy(ov, out_hbm.at[pl.ds(rbase * jnp.int32(_F), _CH * _F)])
        return carry

    lax.fori_loop(jnp.int32(0), jnp.int32(_RPW // _CH), do_chunk, jnp.int32(0))


_mesh = plsc.VectorSubcoreMesh(core_axis_name="c", subcore_axis_name="s")

_sc_kernel = functools.partial(
    pl.kernel,
    out_type=jax.ShapeDtypeStruct((_N * _F,), jnp.float32),
    mesh=_mesh,
    compiler_params=pltpu.CompilerParams(needs_layout_passes=False),
    scratch_types=[
        pltpu.VMEM((_K * _F,), jnp.float32),
        pltpu.VMEM((_CH * _K,), jnp.float32),
        pltpu.VMEM((_CH * _F,), jnp.float32),
        pltpu.VMEM((160,), jnp.float32),
        pltpu.VMEM((160,), jnp.int32),
        pltpu.SemaphoreType.DMA,
    ],
)(_sc_body)


def kernel(tj, kernel):
    x = tj[0].astype(jnp.float32)
    xp = jnp.pad(x, ((1, 1), (1, 1), (0, 0)))
    parts = [xp[i:i + _H, j:j + _W, :] for i in range(_KH) for j in range(_KW)]
    patches = jnp.concatenate(parts, axis=-1).reshape(_N * _K)
    J = kernel.reshape(_K * _F).astype(jnp.float32)
    out = _sc_kernel(patches, J)
    return out.reshape(1, _H, _W, _F).astype(jnp.float64)
